# baseline (speedup 1.0000x reference)
import functools

import jax
import jax.numpy as jnp
from jax import lax
from jax.experimental import pallas as pl
from jax.experimental.pallas import tpu as pltpu
from jax.experimental.pallas import tpu_sc as plsc

_B = 4
_N = 2048
_MAXNBR = 32
_SA_LEVELS = [(512, 0.1, 3), (128, 0.2, 64), (32, 0.4, 128), (8, 0.8, 256)]
_ORDER = ['sa1', 'sa2', 'sa3', 'sa4', 'fp4', 'fp3', 'fp2', 'fp1', 'mlp']
_NLAYERS = {'sa1': 3, 'sa2': 3, 'sa3': 3, 'sa4': 3,
            'fp4': 2, 'fp3': 2, 'fp2': 2, 'fp1': 3, 'mlp': 3}


def _fps_batched(px, py, pz, n, m):
    f32 = jnp.float32
    lane_n = jax.lax.broadcasted_iota(jnp.int32, (1, n), 1)
    lane_m = jax.lax.broadcasted_iota(jnp.int32, (1, m), 1)

    def step(i, c):
        dists, qxr, qyr, qzr, oh = c
        lx = jnp.sum(px * oh, axis=1, keepdims=True)
        ly = jnp.sum(py * oh, axis=1, keepdims=True)
        lz = jnp.sum(pz * oh, axis=1, keepdims=True)
        qxr = jnp.where(lane_m == i, lx, qxr)
        qyr = jnp.where(lane_m == i, ly, qyr)
        qzr = jnp.where(lane_m == i, lz, qzr)
        d = (px - lx) ** 2 + (py - ly) ** 2 + (pz - lz) ** 2
        dists = jnp.minimum(dists, d)
        mx = jnp.max(dists, axis=1, keepdims=True)
        nxt = jnp.min(jnp.where(dists == mx, lane_n, n), axis=1, keepdims=True)
        oh = (lane_n == nxt).astype(f32)
        return dists, qxr, qyr, qzr, oh

    zero_m = jnp.zeros((_B, m), f32)
    oh0 = jnp.where(lane_n == 0, jnp.ones((_B, n), f32), jnp.zeros((_B, n), f32))
    init = (jnp.full((_B, n), jnp.inf, f32), zero_m, zero_m, zero_m, oh0)
    out = jax.lax.fori_loop(0, m, step, init)
    return out[1], out[2], out[3]


def _fps_body(px_ref, py_ref, pz_ref, *out_refs):
    px, py, pz = px_ref[...], py_ref[...], pz_ref[...]
    n = _N
    k = 0
    for m, _, _ in _SA_LEVELS:
        qx, qy, qz = _fps_batched(px, py, pz, n, m)
        out_refs[k][...] = qx
        out_refs[k + 1][...] = qy
        out_refs[k + 2][...] = qz
        k += 3
        px, py, pz = qx, qy, qz
        n = m


def _sel_body(pt_ref, qc_ref, idx_ref, valid_ref):
    m, r, _ = _SA_LEVELS[0]
    n = _N
    px = pt_ref[0:1, :]
    py = pt_ref[1:2, :]
    pz = pt_ref[2:3, :]
    q3 = qc_ref[...]
    qx, qy, qz = q3[:, 0:1], q3[:, 1:2], q3[:, 2:3]
    D = (qx - px) ** 2 + (qy - py) ** 2 + (qz - pz) ** 2
    Dm0 = jnp.where(D <= r * r, D, jnp.inf)
    lane_n = jax.lax.broadcasted_iota(jnp.int32, (1, n), 1)
    lane_k = jax.lax.broadcasted_iota(jnp.int32, (1, _MAXNBR), 1)

    def jcond(carry):
        j, _, _, _, alive = carry
        return jnp.logical_and(j < _MAXNBR, alive)

    def jstep(carry):
        j, Dm, idxb, vb, _ = carry
        rmin = jnp.min(Dm, axis=1, keepdims=True)
        alive = jnp.min(rmin) < jnp.inf
        sidx = jnp.min(jnp.where(Dm == rmin, lane_n, n), axis=1, keepdims=True)
        Dm = jnp.where(lane_n == sidx, jnp.inf, Dm)
        idxb = jnp.where(lane_k == j, sidx, idxb)
        vb = jnp.where(lane_k == j, (rmin < jnp.inf).astype(jnp.float32), vb)
        return j + 1, Dm, idxb, vb, alive

    init = (jnp.int32(0), Dm0, jnp.zeros((m, _MAXNBR), jnp.int32),
            jnp.zeros((m, _MAXNBR), jnp.float32), jnp.bool_(True))
    out = jax.lax.while_loop(jcond, jstep, init)
    idx_ref[...] = out[2] + pl.program_id(0) * _N
    valid_ref[...] = out[3]


def _sc_gather(tbl, idx):
    info = plsc.get_sparse_core_info()
    nc, ns = info.num_cores, info.num_subcores
    nw = nc * ns
    btot = idx.shape[0]
    d = tbl.shape[1]
    b_per_w = btot // nw
    chunk = 128
    nchunks = b_per_w // chunk
    mesh = plsc.VectorSubcoreMesh(core_axis_name="c", subcore_axis_name="s")

    @functools.partial(
        pl.kernel, mesh=mesh,
        out_type=jax.ShapeDtypeStruct((btot, d), jnp.float32),
        scratch_types=[pltpu.VMEM((chunk,), jnp.int32),
                       pltpu.VMEM((chunk, d), jnp.float32),
                       pltpu.SemaphoreType.DMA],
    )
    def k(tbl_hbm, idx_hbm, out_hbm, idx_v, rows_v, sem):
        wid = lax.axis_index("s") * nc + lax.axis_index("c")
        base = wid * b_per_w

        def body(ci, carry):
            off = base + ci * chunk
            pltpu.sync_copy(idx_hbm.at[pl.ds(off, chunk)], idx_v)
            pltpu.async_copy(tbl_hbm.at[idx_v], rows_v, sem).wait()
            pltpu.sync_copy(rows_v, out_hbm.at[pl.ds(off, chunk)])
            return carry

        lax.fori_loop(0, nchunks, body, 0)

    return k(tbl, idx)


def _apply(layers, h):
    for wt, b, act in layers:
        h = jnp.dot(h, wt, preferred_element_type=jnp.float32) + b
        if act:
            h = jnp.maximum(h, 0.0)
    return h


def _sa(xc, pc, px, py, pz, qx, qy, qz, layers, m, n, r, c, cout):
    D = (qx - px) ** 2 + (qy - py) ** 2 + (qz - pz) ** 2
    Dm = jnp.where(D <= r * r, D, jnp.inf)
    feat = jnp.concatenate([xc, pc], axis=1)
    q3 = jnp.concatenate([qx, qy, qz], axis=1)
    lane_n = jax.lax.broadcasted_iota(jnp.int32, (1, n), 1)

    def jcond(carry):
        j, _, _, alive = carry
        return jnp.logical_and(j < _MAXNBR, alive)

    def jstep(carry):
        j, Dm, acc, _ = carry
        rmin = jnp.min(Dm, axis=1, keepdims=True)
        alive = jnp.min(rmin) < jnp.inf
        sidx = jnp.min(jnp.where(Dm == rmin, lane_n, n), axis=1, keepdims=True)
        sel = (lane_n == sidx).astype(jnp.float32)
        Dm = jnp.where(sel > 0, jnp.inf, Dm)
        g = jnp.dot(sel, feat, preferred_element_type=jnp.float32)
        inp = jnp.concatenate([g[:, :c], g[:, c:c + 3] - q3], axis=1)
        h = _apply(layers, inp)
        valid = rmin < jnp.inf
        acc = jnp.maximum(acc, jnp.where(valid, h, -jnp.inf))
        return j + 1, Dm, acc, alive

    acc0 = jnp.full((m, cout), -jnp.inf, jnp.float32)
    out = jax.lax.while_loop(jcond, jstep,
                             (jnp.int32(0), Dm, acc0, jnp.bool_(True)))
    return out[2]


def _fp(xcoarse, xskip, fx, fy, fz, cxr, cyr, czr, layers, nf, nc):
    D = (fx - cxr) ** 2 + (fy - cyr) ** 2 + (fz - czr) ** 2
    lane_c = jax.lax.broadcasted_iota(jnp.int32, (1, nc), 1)
    Wm = jnp.zeros((nf, nc), jnp.float32)
    wsum = jnp.zeros((nf, 1), jnp.float32)
    for _ in range(3):
        rmin = jnp.min(D, axis=1, keepdims=True)
        sidx = jnp.min(jnp.where(D == rmin, lane_c, nc), axis=1, keepdims=True)
        sel = (lane_c == sidx).astype(jnp.float32)
        w = 1.0 / jnp.maximum(rmin, 1e-16)
        Wm = Wm + sel * w
        wsum = wsum + w
        D = jnp.where(sel > 0, jnp.inf, D)
    Wm = Wm / wsum
    xi = jnp.dot(Wm, xcoarse, preferred_element_type=jnp.float32)
    return _apply(layers, jnp.concatenate([xi, xskip], axis=1))


def _body(x_ref, pos_ref, g_ref, v_ref, qc1, qr1, qc2, qr2, qc3, qr3,
          qc4, qr4, *rest):
    out_ref = rest[-1]
    wrefs = rest[:-1]
    layers = {}
    k = 0
    for name in _ORDER:
        ls = []
        for i in range(_NLAYERS[name]):
            act = not (name == 'mlp' and i == _NLAYERS[name] - 1)
            ls.append((wrefs[k][...], wrefs[k + 1][...], act))
            k += 2
        layers[name] = ls

    xc = x_ref[...]
    pc = pos_ref[...]

    qcols = [qc1[...], qc2[...], qc3[...], qc4[...]]
    qrows = [qr1[...], qr2[...], qr3[...], qr4[...]]

    m1 = _SA_LEVELS[0][0]
    q3_1 = qcols[0]
    cout1 = layers['sa1'][-1][0].shape[1]
    x1 = jnp.full((m1, cout1), -jnp.inf, jnp.float32)
    grp = 8
    q3t = jnp.concatenate([q3_1] * grp, axis=0)
    for gi in range(_MAXNBR // grp):
        base = gi * grp * m1
        gg = g_ref[base:base + grp * m1, :]
        vv = v_ref[base:base + grp * m1, :]
        inp = jnp.concatenate([gg[:, 0:3], gg[:, 3:6] - q3t], axis=1)
        h = _apply(layers['sa1'], inp)
        h = jnp.where(vv > 0, h, -jnp.inf)
        for j in range(grp):
            x1 = jnp.maximum(x1, h[j * m1:(j + 1) * m1, :])

    feats = [xc, x1]
    cur_x, cur_pc = x1, q3_1
    cur_px = qrows[0][0:1, :]
    cur_py = qrows[0][1:2, :]
    cur_pz = qrows[0][2:3, :]
    n = m1
    for li, (m, r, c) in enumerate(_SA_LEVELS):
        if li == 0:
            continue
        q3 = qcols[li]
        qx, qy, qz = q3[:, 0:1], q3[:, 1:2], q3[:, 2:3]
        cout = layers['sa%d' % (li + 1)][-1][0].shape[1]
        xo = _sa(cur_x, cur_pc, cur_px, cur_py, cur_pz, qx, qy, qz,
                 layers['sa%d' % (li + 1)], m, n, r, c, cout)
        feats.append(xo)
        cur_x = xo
        cur_pc = q3
        cur_px = qrows[li][0:1, :]
        cur_py = qrows[li][1:2, :]
        cur_pz = qrows[li][2:3, :]
        n = m

    f = feats[4]
    for li in range(4, 0, -1):
        nc = _SA_LEVELS[li - 1][0]
        if li - 2 >= 0:
            nf = _SA_LEVELS[li - 2][0]
            q3f = qcols[li - 2]
            fx, fy, fz = q3f[:, 0:1], q3f[:, 1:2], q3f[:, 2:3]
            xskip = feats[li - 1]
        else:
            nf = _N
            fx, fy, fz = pc[:, 0:1], pc[:, 1:2], pc[:, 2:3]
            xskip = xc
        qr = qrows[li - 1]
        f = _fp(f, xskip, fx, fy, fz, qr[0:1, :], qr[1:2, :], qr[2:3, :],
                layers['fp%d' % li], nf, nc)

    y = _apply(layers['mlp'], f)
    out_ref[...] = y


def kernel(x, pos, batch, params):
    del batch
    inv = 1.0 / jnp.sqrt(1.0 + 1e-5)
    flat = []
    for name in _ORDER:
        ps = params[name]
        for i, (W, b, g, be) in enumerate(ps):
            if name == 'mlp' and i == len(ps) - 1:
                flat += [W.T, b[None, :]]
            else:
                s = g * inv
                flat += [(W * s[:, None]).T, (b * s + be)[None, :]]

    xb = x.reshape(_B, _N, x.shape[-1])
    pb = pos.reshape(_B, _N, 3)
    pt = jnp.transpose(pb, (0, 2, 1))

    fps_out = pl.pallas_call(
        _fps_body,
        out_shape=[jax.ShapeDtypeStruct((_B, m), jnp.float32)
                   for m, _, _ in _SA_LEVELS for _c in range(3)],
    )(pb[:, :, 0], pb[:, :, 1], pb[:, :, 2])

    qcols, qrows = [], []
    for li in range(4):
        qx, qy, qz = fps_out[3 * li], fps_out[3 * li + 1], fps_out[3 * li + 2]
        q3 = jnp.stack([qx, qy, qz], axis=-1)
        qcols.append(q3)
        qrows.append(jnp.transpose(q3, (0, 2, 1)))

    m1 = _SA_LEVELS[0][0]
    idxg, val = pl.pallas_call(
        _sel_body,
        grid=(_B,),
        compiler_params=pltpu.CompilerParams(
            dimension_semantics=("parallel",)),
        in_specs=[pl.BlockSpec((None, 3, _N), lambda b: (b, 0, 0)),
                  pl.BlockSpec((None, m1, 3), lambda b: (b, 0, 0))],
        out_specs=[pl.BlockSpec((None, m1, _MAXNBR), lambda b: (b, 0, 0)),
                   pl.BlockSpec((None, m1, _MAXNBR), lambda b: (b, 0, 0))],
        out_shape=[jax.ShapeDtypeStruct((_B, m1, _MAXNBR), jnp.int32),
                   jax.ShapeDtypeStruct((_B, m1, _MAXNBR), jnp.float32)],
    )(pt, qcols[0])

    tbl = jnp.concatenate(
        [xb.reshape(_B * _N, -1), pb.reshape(_B * _N, 3),
         jnp.zeros((_B * _N, 122), jnp.float32)], axis=1)
    idx_flat = jnp.transpose(idxg, (0, 2, 1)).reshape(_B * _MAXNBR * m1)
    g_rows = _sc_gather(tbl, idx_flat)
    g_all = g_rows[:, :8].reshape(_B, _MAXNBR * m1, 8)
    val_col = jnp.transpose(val, (0, 2, 1)).reshape(_B, _MAXNBR * m1, 1)

    in_specs = [
        pl.BlockSpec((None, _N, xb.shape[-1]), lambda b: (b, 0, 0)),
        pl.BlockSpec((None, _N, 3), lambda b: (b, 0, 0)),
        pl.BlockSpec((None, _MAXNBR * m1, 8), lambda b: (b, 0, 0)),
        pl.BlockSpec((None, _MAXNBR * m1, 1), lambda b: (b, 0, 0)),
    ]
    qargs = []
    for li, (m, _, _) in enumerate(_SA_LEVELS):
        in_specs.append(pl.BlockSpec((None, m, 3), lambda b: (b, 0, 0)))
        in_specs.append(pl.BlockSpec((None, 3, m), lambda b: (b, 0, 0)))
        qargs += [qcols[li], qrows[li]]
    for a in flat:
        in_specs.append(pl.BlockSpec(a.shape, lambda b: (0, 0)))

    out = pl.pallas_call(
        _body,
        grid=(_B,),
        compiler_params=pltpu.CompilerParams(
            dimension_semantics=("parallel",)),
        in_specs=in_specs,
        out_specs=pl.BlockSpec((None, _N, 13), lambda b: (b, 0, 0)),
        out_shape=jax.ShapeDtypeStruct((_B, _N, 13), jnp.float32),
    )(xb, pb, g_all, val_col, *qargs, *flat)
    return out.reshape(_B * _N, 13)


# speedup vs baseline: 1.6422x; 1.6422x over previous
import jax
import jax.numpy as jnp
from jax.experimental import pallas as pl
from jax.experimental.pallas import tpu as pltpu

_B = 4
_N = 2048
_MAXNBR = 32
_SA_LEVELS = [(512, 0.1, 3), (128, 0.2, 64), (32, 0.4, 128), (8, 0.8, 256)]
_ORDER = ['sa1', 'sa2', 'sa3', 'sa4', 'fp4', 'fp3', 'fp2', 'fp1', 'mlp']
_NLAYERS = {'sa1': 3, 'sa2': 3, 'sa3': 3, 'sa4': 3,
            'fp4': 2, 'fp3': 2, 'fp2': 2, 'fp1': 3, 'mlp': 3}


def _fps_batched(px, py, pz, n, m):
    f32 = jnp.float32
    lane_n = jax.lax.broadcasted_iota(jnp.int32, (1, n), 1)
    lane_m = jax.lax.broadcasted_iota(jnp.int32, (1, m), 1)

    def step(i, c):
        dists, qxr, qyr, qzr, oh = c
        lx = jnp.sum(px * oh, axis=1, keepdims=True)
        ly = jnp.sum(py * oh, axis=1, keepdims=True)
        lz = jnp.sum(pz * oh, axis=1, keepdims=True)
        qxr = jnp.where(lane_m == i, lx, qxr)
        qyr = jnp.where(lane_m == i, ly, qyr)
        qzr = jnp.where(lane_m == i, lz, qzr)
        d = (px - lx) ** 2 + (py - ly) ** 2 + (pz - lz) ** 2
        dists = jnp.minimum(dists, d)
        mx = jnp.max(dists, axis=1, keepdims=True)
        nxt = jnp.min(jnp.where(dists == mx, lane_n, n), axis=1, keepdims=True)
        oh = (lane_n == nxt).astype(f32)
        return dists, qxr, qyr, qzr, oh

    zero_m = jnp.zeros((_B, m), f32)
    oh0 = jnp.where(lane_n == 0, jnp.ones((_B, n), f32), jnp.zeros((_B, n), f32))
    init = (jnp.full((_B, n), jnp.inf, f32), zero_m, zero_m, zero_m, oh0)
    out = jax.lax.fori_loop(0, m, step, init)
    return out[1], out[2], out[3]


def _fps_body(px_ref, py_ref, pz_ref, *out_refs):
    px, py, pz = px_ref[...], py_ref[...], pz_ref[...]
    n = _N
    k = 0
    for m, _, _ in _SA_LEVELS:
        qx, qy, qz = _fps_batched(px, py, pz, n, m)
        out_refs[k][...] = qx
        out_refs[k + 1][...] = qy
        out_refs[k + 2][...] = qz
        k += 3
        px, py, pz = qx, qy, qz
        n = m


def _apply(layers, h):
    for wt, b, act in layers:
        h = jnp.dot(h, wt, preferred_element_type=jnp.float32) + b
        if act:
            h = jnp.maximum(h, 0.0)
    return h


def _sa(xc, pc, px, py, pz, qx, qy, qz, layers, m, n, r, c, cout):
    D = (qx - px) ** 2 + (qy - py) ** 2 + (qz - pz) ** 2
    Dm = jnp.where(D <= r * r, D, jnp.inf)
    feat = jnp.concatenate([xc, pc], axis=1)
    q3 = jnp.concatenate([qx, qy, qz], axis=1)
    lane_n = jax.lax.broadcasted_iota(jnp.int32, (1, n), 1)

    def jcond(carry):
        j, _, _, alive = carry
        return jnp.logical_and(j < _MAXNBR, alive)

    def jstep(carry):
        j, Dm, acc, _ = carry
        rmin = jnp.min(Dm, axis=1, keepdims=True)
        alive = jnp.min(rmin) < jnp.inf
        sidx = jnp.min(jnp.where(Dm == rmin, lane_n, n), axis=1, keepdims=True)
        sel = (lane_n == sidx).astype(jnp.float32)
        Dm = jnp.where(sel > 0, jnp.inf, Dm)
        g = jnp.dot(sel, feat, preferred_element_type=jnp.float32)
        inp = jnp.concatenate([g[:, :c], g[:, c:c + 3] - q3], axis=1)
        h = _apply(layers, inp)
        valid = rmin < jnp.inf
        acc = jnp.maximum(acc, jnp.where(valid, h, -jnp.inf))
        return j + 1, Dm, acc, alive

    acc0 = jnp.full((m, cout), -jnp.inf, jnp.float32)
    out = jax.lax.while_loop(jcond, jstep,
                             (jnp.int32(0), Dm, acc0, jnp.bool_(True)))
    return out[2]


def _fp(xcoarse, xskip, fx, fy, fz, cxr, cyr, czr, layers, nf, nc):
    D = (fx - cxr) ** 2 + (fy - cyr) ** 2 + (fz - czr) ** 2
    lane_c = jax.lax.broadcasted_iota(jnp.int32, (1, nc), 1)
    Wm = jnp.zeros((nf, nc), jnp.float32)
    wsum = jnp.zeros((nf, 1), jnp.float32)
    for _ in range(3):
        rmin = jnp.min(D, axis=1, keepdims=True)
        sidx = jnp.min(jnp.where(D == rmin, lane_c, nc), axis=1, keepdims=True)
        sel = (lane_c == sidx).astype(jnp.float32)
        w = 1.0 / jnp.maximum(rmin, 1e-16)
        Wm = Wm + sel * w
        wsum = wsum + w
        D = jnp.where(sel > 0, jnp.inf, D)
    Wm = Wm / wsum
    xi = jnp.dot(Wm, xcoarse, preferred_element_type=jnp.float32)
    return _apply(layers, jnp.concatenate([xi, xskip], axis=1))


def _body(x_ref, pos_ref, pt_ref, qc1, qr1, qc2, qr2, qc3, qr3, qc4, qr4,
          *rest):
    out_ref = rest[-1]
    wrefs = rest[:-1]
    layers = {}
    k = 0
    for name in _ORDER:
        ls = []
        for i in range(_NLAYERS[name]):
            act = not (name == 'mlp' and i == _NLAYERS[name] - 1)
            ls.append((wrefs[k][...], wrefs[k + 1][...], act))
            k += 2
        layers[name] = ls

    xc = x_ref[...]
    pc = pos_ref[...]
    px = pt_ref[0:1, :]
    py = pt_ref[1:2, :]
    pz = pt_ref[2:3, :]

    qcols = [qc1[...], qc2[...], qc3[...], qc4[...]]
    qrows = [qr1[...], qr2[...], qr3[...], qr4[...]]

    feats = [xc]
    cur_x, cur_pc = xc, pc
    cur_px, cur_py, cur_pz = px, py, pz
    n = _N
    for li, (m, r, c) in enumerate(_SA_LEVELS):
        q3 = qcols[li]
        qx, qy, qz = q3[:, 0:1], q3[:, 1:2], q3[:, 2:3]
        cout = layers['sa%d' % (li + 1)][-1][0].shape[1]
        xo = _sa(cur_x, cur_pc, cur_px, cur_py, cur_pz, qx, qy, qz,
                 layers['sa%d' % (li + 1)], m, n, r, c, cout)
        feats.append(xo)
        cur_x = xo
        cur_pc = q3
        cur_px = qrows[li][0:1, :]
        cur_py = qrows[li][1:2, :]
        cur_pz = qrows[li][2:3, :]
        n = m

    f = feats[4]
    for li in range(4, 0, -1):
        nc = _SA_LEVELS[li - 1][0]
        if li - 2 >= 0:
            nf = _SA_LEVELS[li - 2][0]
            q3f = qcols[li - 2]
            fx, fy, fz = q3f[:, 0:1], q3f[:, 1:2], q3f[:, 2:3]
            xskip = feats[li - 1]
        else:
            nf = _N
            fx, fy, fz = pc[:, 0:1], pc[:, 1:2], pc[:, 2:3]
            xskip = xc
        qr = qrows[li - 1]
        f = _fp(f, xskip, fx, fy, fz, qr[0:1, :], qr[1:2, :], qr[2:3, :],
                layers['fp%d' % li], nf, nc)

    y = _apply(layers['mlp'], f)
    out_ref[...] = y


def kernel(x, pos, batch, params):
    del batch
    inv = 1.0 / jnp.sqrt(1.0 + 1e-5)
    flat = []
    for name in _ORDER:
        ps = params[name]
        for i, (W, b, g, be) in enumerate(ps):
            if name == 'mlp' and i == len(ps) - 1:
                flat += [W.T, b[None, :]]
            else:
                s = g * inv
                flat += [(W * s[:, None]).T, (b * s + be)[None, :]]

    xb = x.reshape(_B, _N, x.shape[-1])
    pb = pos.reshape(_B, _N, 3)
    pt = jnp.transpose(pb, (0, 2, 1))

    fps_out = pl.pallas_call(
        _fps_body,
        out_shape=[jax.ShapeDtypeStruct((_B, m), jnp.float32)
                   for m, _, _ in _SA_LEVELS for _c in range(3)],
    )(pb[:, :, 0], pb[:, :, 1], pb[:, :, 2])

    qcols, qrows = [], []
    for li in range(4):
        qx, qy, qz = fps_out[3 * li], fps_out[3 * li + 1], fps_out[3 * li + 2]
        q3 = jnp.stack([qx, qy, qz], axis=-1)
        qcols.append(q3)
        qrows.append(jnp.transpose(q3, (0, 2, 1)))

    in_specs = [
        pl.BlockSpec((None, _N, xb.shape[-1]), lambda b: (b, 0, 0)),
        pl.BlockSpec((None, _N, 3), lambda b: (b, 0, 0)),
        pl.BlockSpec((None, 3, _N), lambda b: (b, 0, 0)),
    ]
    qargs = []
    for li, (m, _, _) in enumerate(_SA_LEVELS):
        in_specs.append(pl.BlockSpec((None, m, 3), lambda b: (b, 0, 0)))
        in_specs.append(pl.BlockSpec((None, 3, m), lambda b: (b, 0, 0)))
        qargs += [qcols[li], qrows[li]]
    for a in flat:
        in_specs.append(pl.BlockSpec(a.shape, lambda b: (0, 0)))

    out = pl.pallas_call(
        _body,
        grid=(_B,),
        compiler_params=pltpu.CompilerParams(
            dimension_semantics=("parallel",)),
        in_specs=in_specs,
        out_specs=pl.BlockSpec((None, _N, 13), lambda b: (b, 0, 0)),
        out_shape=jax.ShapeDtypeStruct((_B, _N, 13), jnp.float32),
    )(xb, pb, pt, *qargs, *flat)
    return out.reshape(_B * _N, 13)
